# baseline (speedup 1.0000x reference)
import functools

import jax
import jax.numpy as jnp
from jax import lax
from jax.experimental import pallas as pl
from jax.experimental.pallas import tpu as pltpu
from jax.experimental.pallas import tpu_sc as plsc

N = 10000
E = 320000
D = 128

NC = 2
NS = 16
NW = NC * NS
L = 16

EPW = E // NW
K = 80
NCHUNK = EPW // K
RPAD = 10240
RPT = RPAD // NS

_mesh = plsc.VectorSubcoreMesh(core_axis_name="c", subcore_axis_name="s")


@functools.partial(
    pl.kernel,
    out_type=jax.ShapeDtypeStruct((NC, RPAD, D), jnp.float32),
    mesh=_mesh,
    scratch_types=[
        pltpu.VMEM((4, K), jnp.int32),
        pltpu.VMEM((K, D), jnp.float32),
        pltpu.VMEM_SHARED((RPAD, D), jnp.float32),
        pltpu.SemaphoreType.DMA((4,)),
        pltpu.SemaphoreType.DMA((4,)),
    ],
)
def _sc_count(dst_hbm, out_hbm, dst_v, ones_v, acc, isem, ssem):
    cid = lax.axis_index("c")
    sid = lax.axis_index("s")
    wid = sid * NC + cid
    base = wid * EPW
    tile_row0 = sid * RPT

    NBC = 4

    def issue_idx(i, b):
        pltpu.async_copy(dst_hbm.at[pl.ds(base + i * K, K)], dst_v.at[b],
                         isem.at[b])

    def wait_idx(b):
        pltpu.make_async_copy(dst_hbm.at[pl.ds(0, K)], dst_v.at[b],
                              isem.at[b]).wait()

    def issue_scatter(b):
        pltpu.async_copy(ones_v, acc.at[dst_v.at[b]], ssem.at[b], add=True)

    def wait_scatter(b):
        pltpu.make_async_copy(ones_v, acc.at[dst_v.at[b]], ssem.at[b]).wait()

    def zrow(r, _):
        for c in range(D // L):
            ones_v[r, pl.ds(c * L, L)] = jnp.zeros((L,), jnp.float32)
        return 0

    lax.fori_loop(0, K, zrow, 0)

    def zero_chunk(j, _):
        pltpu.sync_copy(ones_v, acc.at[pl.ds(tile_row0 + j * K, K)])
        return 0

    lax.fori_loop(0, RPT // K, zero_chunk, 0)

    def fill(r, _):
        for c in range(D // L):
            ones_v[r, pl.ds(c * L, L)] = jnp.ones((L,), jnp.float32)
        return 0

    lax.fori_loop(0, K, fill, 0)
    plsc.subcore_barrier()

    issue_idx(NPIPE, 0)
    wait_idx(0)
    issue_scatter(0)
    wait_scatter(0)

    for b in range(NBC - 1):
        issue_idx(b, b)

    @pl.loop(0, NPIPE // NBC)
    def _outer(j):
        for b in range(NBC):
            i = j * NBC + b
            wait_idx(b)
            issue_scatter(b)

            b3 = (b + 3) % NBC
            fits = i + 3 < NPIPE

            @pl.when(jnp.logical_and(fits, i >= 1))
            def _i():
                wait_scatter(b3)
                issue_idx(i + 3, b3)

            @pl.when(jnp.logical_and(fits, i < 1))
            def _i0():
                issue_idx(i + 3, b3)

    for b in range(NBC):
        wait_scatter(b)

    plsc.subcore_barrier()
    pltpu.sync_copy(acc.at[pl.ds(tile_row0, RPT)],
                    out_hbm.at[cid, pl.ds(tile_row0, RPT)])


NB = 4
NPIPE = NCHUNK - 1


@functools.partial(
    pl.kernel,
    out_type=jax.ShapeDtypeStruct((NC, RPAD, D), jnp.float32),
    mesh=_mesh,
    scratch_types=[
        pltpu.VMEM((NB, K), jnp.int32),
        pltpu.VMEM((NB, K), jnp.int32),
        pltpu.VMEM((NB, K, D), jnp.float32),
        pltpu.VMEM_SHARED((RPAD, D), jnp.float32),
        pltpu.SemaphoreType.DMA((NB,)),
        pltpu.SemaphoreType.DMA((NB,)),
        pltpu.SemaphoreType.DMA((NB,)),
    ],
)
def _sc_aggregate(x_hbm, src_hbm, dst_hbm, out_hbm,
                  src_v, dst_v, rows_v, acc, isem, gsem, ssem):
    cid = lax.axis_index("c")
    sid = lax.axis_index("s")
    wid = sid * NC + cid
    base = wid * EPW
    tile_row0 = sid * RPT

    def issue_idx(i, b):
        off = base + i * K
        pltpu.async_copy(src_hbm.at[pl.ds(off, K)], src_v.at[b], isem.at[b])
        pltpu.async_copy(dst_hbm.at[pl.ds(off, K)], dst_v.at[b], isem.at[b])

    def wait_idx(b):
        pltpu.make_async_copy(src_hbm.at[pl.ds(0, K)], src_v.at[b], isem.at[b]).wait()
        pltpu.make_async_copy(dst_hbm.at[pl.ds(0, K)], dst_v.at[b], isem.at[b]).wait()

    def issue_gather(b):
        pltpu.async_copy(x_hbm.at[src_v.at[b]], rows_v.at[b], gsem.at[b])

    def wait_gather(b):
        pltpu.make_async_copy(x_hbm.at[src_v.at[b]], rows_v.at[b], gsem.at[b]).wait()

    def issue_scatter(b):
        pltpu.async_copy(rows_v.at[b], acc.at[dst_v.at[b]], ssem.at[b], add=True)

    def wait_scatter(b):
        pltpu.make_async_copy(rows_v.at[b], acc.at[dst_v.at[b]], ssem.at[b]).wait()

    def zrow(r, _):
        for c in range(D // L):
            rows_v[0, r, pl.ds(c * L, L)] = jnp.zeros((L,), jnp.float32)
        return 0

    lax.fori_loop(0, K, zrow, 0)

    def zero_chunk(j, _):
        pltpu.sync_copy(rows_v.at[0], acc.at[pl.ds(tile_row0 + j * K, K)])
        return 0

    lax.fori_loop(0, RPT // K, zero_chunk, 0)
    plsc.subcore_barrier()

    issue_idx(NPIPE, 0)
    wait_idx(0)
    issue_gather(0)
    wait_gather(0)
    issue_scatter(0)
    wait_scatter(0)

    for b in range(NB - 1):
        issue_idx(b, b)
    for b in range(2):
        wait_idx(b)
        issue_gather(b)

    @pl.loop(0, NPIPE // NB)
    def _outer(j):
        for b in range(NB):
            i = j * NB + b
            wait_gather(b)
            issue_scatter(b)

            b2 = (b + 2) % NB

            @pl.when(i + 2 < NPIPE)
            def _g():
                wait_idx(b2)
                issue_gather(b2)

            b3 = (b + 3) % NB
            fits = i + 3 < NPIPE

            @pl.when(jnp.logical_and(fits, i >= 1))
            def _i():
                wait_scatter(b3)
                issue_idx(i + 3, b3)

            @pl.when(jnp.logical_and(fits, i < 1))
            def _i0():
                issue_idx(i + 3, b3)

    for b in range(NB):
        wait_scatter(b)

    plsc.subcore_barrier()
    pltpu.sync_copy(acc.at[pl.ds(tile_row0, RPT)],
                    out_hbm.at[cid, pl.ds(tile_row0, RPT)])


BLK = 400
NBLK = N // BLK


def _rcp_body(c_ref, o_ref):
    cnt = c_ref[0, :, :L] + c_ref[1, :, :L]
    o_ref[...] = 1.0 / jnp.maximum(cnt, 1.0)


def _tc_rcp(c2):
    return pl.pallas_call(
        _rcp_body,
        grid=(NBLK,),
        in_specs=[pl.BlockSpec((NC, BLK, D), lambda i: (0, i, 0))],
        out_specs=pl.BlockSpec((BLK, L), lambda i: (i, 0)),
        out_shape=jax.ShapeDtypeStruct((N, L), jnp.float32),
    )(c2)


def _dense_body(relu, p_ref, r_ref, x_ref, wl_ref, wr_ref, b_ref, o_ref):
    mean = (p_ref[0] + p_ref[1]) * r_ref[:, 0][:, None]
    acc = jnp.dot(mean, wl_ref[...], preferred_element_type=jnp.float32,
                  precision=jax.lax.Precision.HIGHEST)
    acc = acc + jnp.dot(x_ref[...], wr_ref[...],
                        preferred_element_type=jnp.float32,
                        precision=jax.lax.Precision.HIGHEST)
    acc = acc + b_ref[...]
    if relu:
        acc = jnp.maximum(acc, 0.0)
    o_ref[...] = acc


def _tc_dense(p, rcp, x, wlT, wrT, b2, relu):
    body = functools.partial(_dense_body, relu)
    return pl.pallas_call(
        body,
        grid=(NBLK,),
        in_specs=[
            pl.BlockSpec((NC, BLK, D), lambda i: (0, i, 0)),
            pl.BlockSpec((BLK, L), lambda i: (i, 0)),
            pl.BlockSpec((BLK, D), lambda i: (i, 0)),
            pl.BlockSpec((D, D), lambda i: (0, 0)),
            pl.BlockSpec((D, D), lambda i: (0, 0)),
            pl.BlockSpec((1, D), lambda i: (0, 0)),
        ],
        out_specs=pl.BlockSpec((BLK, D), lambda i: (i, 0)),
        out_shape=jax.ShapeDtypeStruct((N, D), jnp.float32),
    )(p, rcp, x, wlT, wrT, b2)


def kernel(x, edge_index, Wl0, bl0, Wr0, Wl1, bl1, Wr1, Wl2, bl2, Wr2):
    src = edge_index[0]
    dst = edge_index[1]
    c2 = _sc_count(dst)
    rcp = _tc_rcp(c2)
    h = x
    for (Wl, bl, Wr, relu) in ((Wl0, bl0, Wr0, True),
                               (Wl1, bl1, Wr1, True),
                               (Wl2, bl2, Wr2, False)):
        p = _sc_aggregate(h, src, dst)
        h = _tc_dense(p, rcp, h, Wl.T, Wr.T, bl.reshape(1, D), relu)
    return h


# speedup vs baseline: 1.0246x; 1.0246x over previous
import functools

import jax
import jax.numpy as jnp
from jax import lax
from jax.experimental import pallas as pl
from jax.experimental.pallas import tpu as pltpu
from jax.experimental.pallas import tpu_sc as plsc

N = 10000
E = 320000
D = 128

NC = 2
NS = 16
NW = NC * NS
L = 16

EPW = E // NW
K = 80
NCHUNK = EPW // K
RPAD = 10240
RPT = RPAD // NS

_mesh = plsc.VectorSubcoreMesh(core_axis_name="c", subcore_axis_name="s")


@functools.partial(
    pl.kernel,
    out_type=jax.ShapeDtypeStruct((NC, RPAD, D), jnp.float32),
    mesh=_mesh,
    scratch_types=[
        pltpu.VMEM((4, K), jnp.int32),
        pltpu.VMEM((K, D), jnp.float32),
        pltpu.VMEM_SHARED((RPAD, D), jnp.float32),
        pltpu.SemaphoreType.DMA((4,)),
        pltpu.SemaphoreType.DMA((4,)),
    ],
)
def _sc_count(ei_hbm, out_hbm, dst_v, ones_v, acc, isem, ssem):
    cid = lax.axis_index("c")
    sid = lax.axis_index("s")
    wid = sid * NC + cid
    base = wid * EPW
    tile_row0 = sid * RPT

    NBC = 4

    def issue_idx(i, b):
        pltpu.async_copy(ei_hbm.at[pl.ds(E + base + i * K, K)], dst_v.at[b],
                         isem.at[b])

    def wait_idx(b):
        pltpu.make_async_copy(ei_hbm.at[pl.ds(0, K)], dst_v.at[b],
                              isem.at[b]).wait()

    def issue_scatter(b):
        pltpu.async_copy(ones_v, acc.at[dst_v.at[b]], ssem.at[b], add=True)

    def wait_scatter(b):
        pltpu.make_async_copy(ones_v, acc.at[dst_v.at[b]], ssem.at[b]).wait()

    def zrow(r, _):
        for c in range(D // L):
            ones_v[r, pl.ds(c * L, L)] = jnp.zeros((L,), jnp.float32)
        return 0

    lax.fori_loop(0, K, zrow, 0)

    def zero_chunk(j, _):
        pltpu.sync_copy(ones_v, acc.at[pl.ds(tile_row0 + j * K, K)])
        return 0

    lax.fori_loop(0, RPT // K, zero_chunk, 0)

    def fill(r, _):
        for c in range(D // L):
            ones_v[r, pl.ds(c * L, L)] = jnp.ones((L,), jnp.float32)
        return 0

    lax.fori_loop(0, K, fill, 0)
    plsc.subcore_barrier()

    issue_idx(NPIPE, 0)
    wait_idx(0)
    issue_scatter(0)
    wait_scatter(0)

    for b in range(NBC - 1):
        issue_idx(b, b)

    @pl.loop(0, NPIPE // NBC)
    def _outer(j):
        for b in range(NBC):
            i = j * NBC + b
            wait_idx(b)
            issue_scatter(b)

            b3 = (b + 3) % NBC
            fits = i + 3 < NPIPE

            @pl.when(jnp.logical_and(fits, i >= 1))
            def _i():
                wait_scatter(b3)
                issue_idx(i + 3, b3)

            @pl.when(jnp.logical_and(fits, i < 1))
            def _i0():
                issue_idx(i + 3, b3)

    for b in range(NBC):
        wait_scatter(b)

    plsc.subcore_barrier()
    pltpu.sync_copy(acc.at[pl.ds(tile_row0, RPT)],
                    out_hbm.at[cid, pl.ds(tile_row0, RPT)])


NB = 4
NPIPE = NCHUNK - 1


@functools.partial(
    pl.kernel,
    out_type=jax.ShapeDtypeStruct((NC, RPAD, D), jnp.float32),
    mesh=_mesh,
    scratch_types=[
        pltpu.VMEM((NB, K), jnp.int32),
        pltpu.VMEM((NB, K), jnp.int32),
        pltpu.VMEM((NB, K, D), jnp.float32),
        pltpu.VMEM_SHARED((RPAD, D), jnp.float32),
        pltpu.SemaphoreType.DMA((NB,)),
        pltpu.SemaphoreType.DMA((NB,)),
        pltpu.SemaphoreType.DMA((NB,)),
    ],
)
def _sc_aggregate(x_hbm, ei_hbm, out_hbm,
                  src_v, dst_v, rows_v, acc, isem, gsem, ssem):
    cid = lax.axis_index("c")
    sid = lax.axis_index("s")
    wid = sid * NC + cid
    base = wid * EPW
    tile_row0 = sid * RPT

    def issue_idx(i, b):
        off = base + i * K
        pltpu.async_copy(ei_hbm.at[pl.ds(off, K)], src_v.at[b], isem.at[b])
        pltpu.async_copy(ei_hbm.at[pl.ds(E + off, K)], dst_v.at[b], isem.at[b])

    def wait_idx(b):
        pltpu.make_async_copy(ei_hbm.at[pl.ds(0, K)], src_v.at[b], isem.at[b]).wait()
        pltpu.make_async_copy(ei_hbm.at[pl.ds(0, K)], dst_v.at[b], isem.at[b]).wait()

    def issue_gather(b):
        pltpu.async_copy(x_hbm.at[src_v.at[b]], rows_v.at[b], gsem.at[b])

    def wait_gather(b):
        pltpu.make_async_copy(x_hbm.at[src_v.at[b]], rows_v.at[b], gsem.at[b]).wait()

    def issue_scatter(b):
        pltpu.async_copy(rows_v.at[b], acc.at[dst_v.at[b]], ssem.at[b], add=True)

    def wait_scatter(b):
        pltpu.make_async_copy(rows_v.at[b], acc.at[dst_v.at[b]], ssem.at[b]).wait()

    def zrow(r, _):
        for c in range(D // L):
            rows_v[0, r, pl.ds(c * L, L)] = jnp.zeros((L,), jnp.float32)
        return 0

    lax.fori_loop(0, K, zrow, 0)

    def zero_chunk(j, _):
        pltpu.sync_copy(rows_v.at[0], acc.at[pl.ds(tile_row0 + j * K, K)])
        return 0

    lax.fori_loop(0, RPT // K, zero_chunk, 0)
    plsc.subcore_barrier()

    issue_idx(NPIPE, 0)
    wait_idx(0)
    issue_gather(0)
    wait_gather(0)
    issue_scatter(0)
    wait_scatter(0)

    for b in range(NB - 1):
        issue_idx(b, b)
    for b in range(2):
        wait_idx(b)
        issue_gather(b)

    @pl.loop(0, NPIPE // NB)
    def _outer(j):
        for b in range(NB):
            i = j * NB + b
            wait_gather(b)
            issue_scatter(b)

            b2 = (b + 2) % NB

            @pl.when(i + 2 < NPIPE)
            def _g():
                wait_idx(b2)
                issue_gather(b2)

            b3 = (b + 3) % NB
            fits = i + 3 < NPIPE

            @pl.when(jnp.logical_and(fits, i >= 1))
            def _i():
                wait_scatter(b3)
                issue_idx(i + 3, b3)

            @pl.when(jnp.logical_and(fits, i < 1))
            def _i0():
                issue_idx(i + 3, b3)

    for b in range(NB):
        wait_scatter(b)

    plsc.subcore_barrier()
    pltpu.sync_copy(acc.at[pl.ds(tile_row0, RPT)],
                    out_hbm.at[cid, pl.ds(tile_row0, RPT)])


BLK = 400
NBLK = N // BLK


def _rcp_body(c_ref, o_ref):
    cnt = c_ref[0, :, :L] + c_ref[1, :, :L]
    o_ref[...] = 1.0 / jnp.maximum(cnt, 1.0)


def _tc_rcp(c2):
    return pl.pallas_call(
        _rcp_body,
        grid=(NBLK,),
        in_specs=[pl.BlockSpec((NC, BLK, D), lambda i: (0, i, 0))],
        out_specs=pl.BlockSpec((BLK, L), lambda i: (i, 0)),
        out_shape=jax.ShapeDtypeStruct((N, L), jnp.float32),
    )(c2)


_DNUMS = (((1,), (1,)), ((), ()))


def _dense_body(relu, p_ref, r_ref, x_ref, wl_ref, wr_ref, b_ref, o_ref):
    mean = (p_ref[0] + p_ref[1]) * r_ref[:, 0][:, None]
    acc = lax.dot_general(mean, wl_ref[...], _DNUMS,
                          preferred_element_type=jnp.float32,
                          precision=jax.lax.Precision.HIGHEST)
    acc = acc + lax.dot_general(x_ref[...], wr_ref[...], _DNUMS,
                                preferred_element_type=jnp.float32,
                                precision=jax.lax.Precision.HIGHEST)
    acc = acc + b_ref[...]
    if relu:
        acc = jnp.maximum(acc, 0.0)
    o_ref[...] = acc


def _tc_dense(p, rcp, x, wlT, wrT, b2, relu):
    body = functools.partial(_dense_body, relu)
    return pl.pallas_call(
        body,
        grid=(NBLK,),
        in_specs=[
            pl.BlockSpec((NC, BLK, D), lambda i: (0, i, 0)),
            pl.BlockSpec((BLK, L), lambda i: (i, 0)),
            pl.BlockSpec((BLK, D), lambda i: (i, 0)),
            pl.BlockSpec((D, D), lambda i: (0, 0)),
            pl.BlockSpec((D, D), lambda i: (0, 0)),
            pl.BlockSpec((1, D), lambda i: (0, 0)),
        ],
        out_specs=pl.BlockSpec((BLK, D), lambda i: (i, 0)),
        out_shape=jax.ShapeDtypeStruct((N, D), jnp.float32),
    )(p, rcp, x, wlT, wrT, b2)


def kernel(x, edge_index, Wl0, bl0, Wr0, Wl1, bl1, Wr1, Wl2, bl2, Wr2):
    ei = edge_index.reshape(-1)
    c2 = _sc_count(ei)
    p = _sc_aggregate(x, ei)
    rcp = _tc_rcp(c2)
    h = _tc_dense(p, rcp, x, Wl0, Wr0, bl0.reshape(1, D), True)
    p = _sc_aggregate(h, ei)
    h = _tc_dense(p, rcp, h, Wl1, Wr1, bl1.reshape(1, D), True)
    p = _sc_aggregate(h, ei)
    h = _tc_dense(p, rcp, h, Wl2, Wr2, bl2.reshape(1, D), False)
    return h


# speedup vs baseline: 1.1018x; 1.0753x over previous
import functools

import jax
import jax.numpy as jnp
from jax import lax
from jax.experimental import pallas as pl
from jax.experimental.pallas import tpu as pltpu
from jax.experimental.pallas import tpu_sc as plsc

N = 10000
E = 320000
D = 128

NC = 2
NS = 16
NW = NC * NS
L = 16

EPW = E // NW
K = 80
NCHUNK = EPW // K
RPAD = 10240
RPT = RPAD // NS

_mesh = plsc.VectorSubcoreMesh(core_axis_name="c", subcore_axis_name="s")


@functools.partial(
    pl.kernel,
    out_type=jax.ShapeDtypeStruct((NC, RPAD, D), jnp.float32),
    mesh=_mesh,
    scratch_types=[
        pltpu.VMEM((4, K), jnp.int32),
        pltpu.VMEM((K, D), jnp.float32),
        pltpu.VMEM_SHARED((RPAD, D), jnp.float32),
        pltpu.SemaphoreType.DMA((4,)),
        pltpu.SemaphoreType.DMA((4,)),
    ],
)
def _sc_count(ei_hbm, out_hbm, dst_v, ones_v, acc, isem, ssem):
    cid = lax.axis_index("c")
    sid = lax.axis_index("s")
    wid = sid * NC + cid
    base = wid * EPW
    tile_row0 = sid * RPT

    NBC = 4

    def issue_idx(i, b):
        pltpu.async_copy(ei_hbm.at[pl.ds(E + base + i * K, K)], dst_v.at[b],
                         isem.at[b])

    def wait_idx(b):
        pltpu.make_async_copy(ei_hbm.at[pl.ds(0, K)], dst_v.at[b],
                              isem.at[b]).wait()

    def issue_scatter(b):
        pltpu.async_copy(ones_v, acc.at[dst_v.at[b]], ssem.at[b], add=True)

    def wait_scatter(b):
        pltpu.make_async_copy(ones_v, acc.at[dst_v.at[b]], ssem.at[b]).wait()

    def zrow(r, _):
        for c in range(D // L):
            ones_v[r, pl.ds(c * L, L)] = jnp.zeros((L,), jnp.float32)
        return 0

    lax.fori_loop(0, K, zrow, 0)

    def zero_chunk(j, _):
        pltpu.sync_copy(ones_v, acc.at[pl.ds(tile_row0 + j * K, K)])
        return 0

    lax.fori_loop(0, RPT // K, zero_chunk, 0)

    def fill(r, _):
        for c in range(D // L):
            ones_v[r, pl.ds(c * L, L)] = jnp.ones((L,), jnp.float32)
        return 0

    lax.fori_loop(0, K, fill, 0)
    plsc.subcore_barrier()

    issue_idx(NPIPE, 0)
    wait_idx(0)
    issue_scatter(0)
    wait_scatter(0)

    for b in range(NBC - 1):
        issue_idx(b, b)

    @pl.loop(0, NPIPE // NBC)
    def _outer(j):
        for b in range(NBC):
            i = j * NBC + b
            wait_idx(b)
            issue_scatter(b)

            b3 = (b + 3) % NBC
            fits = i + 3 < NPIPE

            @pl.when(jnp.logical_and(fits, i >= 1))
            def _i():
                wait_scatter(b3)
                issue_idx(i + 3, b3)

            @pl.when(jnp.logical_and(fits, i < 1))
            def _i0():
                issue_idx(i + 3, b3)

    for b in range(NBC):
        wait_scatter(b)

    plsc.subcore_barrier()
    pltpu.sync_copy(acc.at[pl.ds(tile_row0, RPT)],
                    out_hbm.at[cid, pl.ds(tile_row0, RPT)])


NB = 4
NPIPE = NCHUNK - 1


@functools.partial(
    pl.kernel,
    out_type=jax.ShapeDtypeStruct((NC, RPAD, D), jnp.float32),
    mesh=_mesh,
    scratch_types=[
        pltpu.VMEM((NB, K), jnp.int32),
        pltpu.VMEM((NB, K), jnp.int32),
        pltpu.VMEM((NB, K, D), jnp.float32),
        pltpu.VMEM_SHARED((RPAD, D), jnp.float32),
        pltpu.SemaphoreType.DMA((NB,)),
        pltpu.SemaphoreType.DMA((NB,)),
        pltpu.SemaphoreType.DMA((NB,)),
    ],
)
def _sc_aggregate(x_hbm, ei_hbm, out_hbm,
                  src_v, dst_v, rows_v, acc, isem, gsem, ssem):
    cid = lax.axis_index("c")
    sid = lax.axis_index("s")
    wid = sid * NC + cid
    base = wid * EPW
    tile_row0 = sid * RPT

    def issue_idx(i, b):
        off = base + i * K
        pltpu.async_copy(ei_hbm.at[pl.ds(off, K)], src_v.at[b], isem.at[b])
        pltpu.async_copy(ei_hbm.at[pl.ds(E + off, K)], dst_v.at[b], isem.at[b])

    def wait_idx(b):
        pltpu.make_async_copy(ei_hbm.at[pl.ds(0, K)], src_v.at[b], isem.at[b]).wait()
        pltpu.make_async_copy(ei_hbm.at[pl.ds(0, K)], dst_v.at[b], isem.at[b]).wait()

    def issue_gather(b):
        pltpu.async_copy(x_hbm.at[src_v.at[b]], rows_v.at[b], gsem.at[b])

    def wait_gather(b):
        pltpu.make_async_copy(x_hbm.at[src_v.at[b]], rows_v.at[b], gsem.at[b]).wait()

    def issue_scatter(b):
        pltpu.async_copy(rows_v.at[b], acc.at[dst_v.at[b]], ssem.at[b], add=True)

    def wait_scatter(b):
        pltpu.make_async_copy(rows_v.at[b], acc.at[dst_v.at[b]], ssem.at[b]).wait()

    def zrow(r, _):
        for c in range(D // L):
            rows_v[0, r, pl.ds(c * L, L)] = jnp.zeros((L,), jnp.float32)
        return 0

    lax.fori_loop(0, K, zrow, 0)

    def zero_chunk(j, _):
        pltpu.sync_copy(rows_v.at[0], acc.at[pl.ds(tile_row0 + j * K, K)])
        return 0

    lax.fori_loop(0, RPT // K, zero_chunk, 0)
    plsc.subcore_barrier()

    issue_idx(NPIPE, 0)
    wait_idx(0)
    issue_gather(0)
    wait_gather(0)
    issue_scatter(0)
    wait_scatter(0)

    for b in range(NB - 1):
        issue_idx(b, b)
    for b in range(2):
        wait_idx(b)
        issue_gather(b)

    @pl.loop(0, NPIPE // NB)
    def _outer(j):
        for b in range(NB):
            i = j * NB + b
            wait_gather(b)
            issue_scatter(b)

            b2 = (b + 2) % NB

            @pl.when(i + 2 < NPIPE)
            def _g():
                wait_idx(b2)
                issue_gather(b2)

            b3 = (b + 3) % NB
            fits = i + 3 < NPIPE

            @pl.when(jnp.logical_and(fits, i >= 1))
            def _i():
                wait_scatter(b3)
                issue_idx(i + 3, b3)

            @pl.when(jnp.logical_and(fits, i < 1))
            def _i0():
                issue_idx(i + 3, b3)

    for b in range(NB):
        wait_scatter(b)

    plsc.subcore_barrier()
    pltpu.sync_copy(acc.at[pl.ds(tile_row0, RPT)],
                    out_hbm.at[cid, pl.ds(tile_row0, RPT)])


BLK = 2000
NBLK = N // BLK
RBLK = 400
NRBLK = N // RBLK


def _rcp_body(c_ref, o_ref):
    cnt = c_ref[0, :, :L] + c_ref[1, :, :L]
    o_ref[...] = 1.0 / jnp.maximum(cnt, 1.0)


def _tc_rcp(c2):
    return pl.pallas_call(
        _rcp_body,
        grid=(NRBLK,),
        in_specs=[pl.BlockSpec((NC, RBLK, D), lambda i: (0, i, 0))],
        out_specs=pl.BlockSpec((RBLK, L), lambda i: (i, 0)),
        out_shape=jax.ShapeDtypeStruct((N, L), jnp.float32),
    )(c2)


_DNUMS = (((1,), (1,)), ((), ()))


def _dense_body(relu, p_ref, r_ref, x_ref, wl_ref, wr_ref, b_ref, o_ref):
    mean = (p_ref[0] + p_ref[1]) * r_ref[:, 0][:, None]
    acc = lax.dot_general(mean, wl_ref[...], _DNUMS,
                          preferred_element_type=jnp.float32,
                          precision=jax.lax.Precision.HIGHEST)
    acc = acc + lax.dot_general(x_ref[...], wr_ref[...], _DNUMS,
                                preferred_element_type=jnp.float32,
                                precision=jax.lax.Precision.HIGHEST)
    acc = acc + b_ref[...]
    if relu:
        acc = jnp.maximum(acc, 0.0)
    o_ref[...] = acc


def _tc_dense(p, rcp, x, wlT, wrT, b2, relu):
    body = functools.partial(_dense_body, relu)
    return pl.pallas_call(
        body,
        grid=(NBLK,),
        in_specs=[
            pl.BlockSpec((NC, BLK, D), lambda i: (0, i, 0)),
            pl.BlockSpec((BLK, L), lambda i: (i, 0)),
            pl.BlockSpec((BLK, D), lambda i: (i, 0)),
            pl.BlockSpec((D, D), lambda i: (0, 0)),
            pl.BlockSpec((D, D), lambda i: (0, 0)),
            pl.BlockSpec((1, D), lambda i: (0, 0)),
        ],
        out_specs=pl.BlockSpec((BLK, D), lambda i: (i, 0)),
        out_shape=jax.ShapeDtypeStruct((N, D), jnp.float32),
    )(p, rcp, x, wlT, wrT, b2)


def kernel(x, edge_index, Wl0, bl0, Wr0, Wl1, bl1, Wr1, Wl2, bl2, Wr2):
    ei = edge_index.reshape(-1)
    c2 = _sc_count(ei)
    p = _sc_aggregate(x, ei)
    rcp = _tc_rcp(c2)
    h = _tc_dense(p, rcp, x, Wl0, Wr0, bl0.reshape(1, D), True)
    p = _sc_aggregate(h, ei)
    h = _tc_dense(p, rcp, h, Wl1, Wr1, bl1.reshape(1, D), True)
    p = _sc_aggregate(h, ei)
    h = _tc_dense(p, rcp, h, Wl2, Wr2, bl2.reshape(1, D), False)
    return h


# speedup vs baseline: 1.1086x; 1.0062x over previous
import functools

import jax
import jax.numpy as jnp
from jax import lax
from jax.experimental import pallas as pl
from jax.experimental.pallas import tpu as pltpu
from jax.experimental.pallas import tpu_sc as plsc

N = 10000
E = 320000
D = 128

NC = 2
NS = 16
NW = NC * NS
L = 16

EPW = E // NW
K = 80
NCHUNK = EPW // K
RPAD = 10240
RPT = RPAD // NS

_mesh = plsc.VectorSubcoreMesh(core_axis_name="c", subcore_axis_name="s")


@functools.partial(
    pl.kernel,
    out_type=jax.ShapeDtypeStruct((NC, RPAD, D), jnp.float32),
    mesh=_mesh,
    scratch_types=[
        pltpu.VMEM((4, K), jnp.int32),
        pltpu.VMEM((K, D), jnp.float32),
        pltpu.VMEM_SHARED((RPAD, D), jnp.float32),
        pltpu.SemaphoreType.DMA((4,)),
        pltpu.SemaphoreType.DMA((4,)),
    ],
)
def _sc_count(ei_hbm, out_hbm, dst_v, ones_v, acc, isem, ssem):
    cid = lax.axis_index("c")
    sid = lax.axis_index("s")
    wid = sid * NC + cid
    base = wid * EPW
    tile_row0 = sid * RPT

    NBC = 4

    def issue_idx(i, b):
        pltpu.async_copy(ei_hbm.at[pl.ds(E + base + i * K, K)], dst_v.at[b],
                         isem.at[b])

    def wait_idx(b):
        pltpu.make_async_copy(ei_hbm.at[pl.ds(0, K)], dst_v.at[b],
                              isem.at[b]).wait()

    def issue_scatter(b):
        pltpu.async_copy(ones_v, acc.at[dst_v.at[b]], ssem.at[b], add=True)

    def wait_scatter(b):
        pltpu.make_async_copy(ones_v, acc.at[dst_v.at[b]], ssem.at[b]).wait()

    issue_idx(NPIPE, 0)

    def zrow(r, _):
        for c in range(D // L):
            ones_v[r, pl.ds(c * L, L)] = jnp.zeros((L,), jnp.float32)
        return 0

    lax.fori_loop(0, K, zrow, 0)

    def zero_chunk(j, _):
        pltpu.sync_copy(ones_v, acc.at[pl.ds(tile_row0 + j * K, K)])
        return 0

    lax.fori_loop(0, RPT // K, zero_chunk, 0)

    def fill(r, _):
        for c in range(D // L):
            ones_v[r, pl.ds(c * L, L)] = jnp.ones((L,), jnp.float32)
        return 0

    lax.fori_loop(0, K, fill, 0)
    plsc.subcore_barrier()

    wait_idx(0)
    issue_scatter(0)
    wait_scatter(0)

    for b in range(NBC - 1):
        issue_idx(b, b)

    @pl.loop(0, NPIPE // NBC)
    def _outer(j):
        for b in range(NBC):
            i = j * NBC + b
            wait_idx(b)
            issue_scatter(b)

            b3 = (b + 3) % NBC
            fits = i + 3 < NPIPE

            @pl.when(jnp.logical_and(fits, i >= 1))
            def _i():
                wait_scatter(b3)
                issue_idx(i + 3, b3)

            @pl.when(jnp.logical_and(fits, i < 1))
            def _i0():
                issue_idx(i + 3, b3)

    for b in range(NBC):
        wait_scatter(b)

    plsc.subcore_barrier()
    pltpu.sync_copy(acc.at[pl.ds(tile_row0, RPT)],
                    out_hbm.at[cid, pl.ds(tile_row0, RPT)])


NB = 4
NPIPE = NCHUNK - 1


@functools.partial(
    pl.kernel,
    out_type=jax.ShapeDtypeStruct((NC, RPAD, D), jnp.float32),
    mesh=_mesh,
    scratch_types=[
        pltpu.VMEM((NB, K), jnp.int32),
        pltpu.VMEM((NB, K), jnp.int32),
        pltpu.VMEM((NB, K, D), jnp.float32),
        pltpu.VMEM_SHARED((RPAD, D), jnp.float32),
        pltpu.SemaphoreType.DMA((NB,)),
        pltpu.SemaphoreType.DMA((NB,)),
        pltpu.SemaphoreType.DMA((NB,)),
    ],
)
def _sc_aggregate(x_hbm, ei_hbm, out_hbm,
                  src_v, dst_v, rows_v, acc, isem, gsem, ssem):
    cid = lax.axis_index("c")
    sid = lax.axis_index("s")
    wid = sid * NC + cid
    base = wid * EPW
    tile_row0 = sid * RPT

    def issue_idx(i, b):
        off = base + i * K
        pltpu.async_copy(ei_hbm.at[pl.ds(off, K)], src_v.at[b], isem.at[b])
        pltpu.async_copy(ei_hbm.at[pl.ds(E + off, K)], dst_v.at[b], isem.at[b])

    def wait_idx(b):
        pltpu.make_async_copy(ei_hbm.at[pl.ds(0, K)], src_v.at[b], isem.at[b]).wait()
        pltpu.make_async_copy(ei_hbm.at[pl.ds(0, K)], dst_v.at[b], isem.at[b]).wait()

    def issue_gather(b):
        pltpu.async_copy(x_hbm.at[src_v.at[b]], rows_v.at[b], gsem.at[b])

    def wait_gather(b):
        pltpu.make_async_copy(x_hbm.at[src_v.at[b]], rows_v.at[b], gsem.at[b]).wait()

    def issue_scatter(b):
        pltpu.async_copy(rows_v.at[b], acc.at[dst_v.at[b]], ssem.at[b], add=True)

    def wait_scatter(b):
        pltpu.make_async_copy(rows_v.at[b], acc.at[dst_v.at[b]], ssem.at[b]).wait()

    issue_idx(NPIPE, 1)
    wait_idx(1)
    issue_gather(1)

    def zrow(r, _):
        for c in range(D // L):
            rows_v[0, r, pl.ds(c * L, L)] = jnp.zeros((L,), jnp.float32)
        return 0

    lax.fori_loop(0, K, zrow, 0)

    def zero_chunk(j, _):
        pltpu.sync_copy(rows_v.at[0], acc.at[pl.ds(tile_row0 + j * K, K)])
        return 0

    lax.fori_loop(0, RPT // K, zero_chunk, 0)
    plsc.subcore_barrier()

    wait_gather(1)
    issue_scatter(1)
    wait_scatter(1)

    for b in range(NB - 1):
        issue_idx(b, b)
    for b in range(2):
        wait_idx(b)
        issue_gather(b)

    @pl.loop(0, NPIPE // NB)
    def _outer(j):
        for b in range(NB):
            i = j * NB + b
            wait_gather(b)
            issue_scatter(b)

            b2 = (b + 2) % NB

            @pl.when(i + 2 < NPIPE)
            def _g():
                wait_idx(b2)
                issue_gather(b2)

            b3 = (b + 3) % NB
            fits = i + 3 < NPIPE

            @pl.when(jnp.logical_and(fits, i >= 1))
            def _i():
                wait_scatter(b3)
                issue_idx(i + 3, b3)

            @pl.when(jnp.logical_and(fits, i < 1))
            def _i0():
                issue_idx(i + 3, b3)

    for b in range(NB):
        wait_scatter(b)

    plsc.subcore_barrier()
    pltpu.sync_copy(acc.at[pl.ds(tile_row0, RPT)],
                    out_hbm.at[cid, pl.ds(tile_row0, RPT)])


BLK = 2000
NBLK = N // BLK
RBLK = 400
NRBLK = N // RBLK


def _rcp_body(c_ref, o_ref):
    cnt = c_ref[0, :, :L] + c_ref[1, :, :L]
    o_ref[...] = 1.0 / jnp.maximum(cnt, 1.0)


def _tc_rcp(c2):
    return pl.pallas_call(
        _rcp_body,
        grid=(NRBLK,),
        in_specs=[pl.BlockSpec((NC, RBLK, D), lambda i: (0, i, 0))],
        out_specs=pl.BlockSpec((RBLK, L), lambda i: (i, 0)),
        out_shape=jax.ShapeDtypeStruct((N, L), jnp.float32),
    )(c2)


_DNUMS = (((1,), (1,)), ((), ()))


def _dense_body(relu, p_ref, r_ref, x_ref, wl_ref, wr_ref, b_ref, o_ref):
    mean = (p_ref[0] + p_ref[1]) * r_ref[:, 0][:, None]
    acc = lax.dot_general(mean, wl_ref[...], _DNUMS,
                          preferred_element_type=jnp.float32,
                          precision=jax.lax.Precision.HIGHEST)
    acc = acc + lax.dot_general(x_ref[...], wr_ref[...], _DNUMS,
                                preferred_element_type=jnp.float32,
                                precision=jax.lax.Precision.HIGHEST)
    acc = acc + b_ref[...]
    if relu:
        acc = jnp.maximum(acc, 0.0)
    o_ref[...] = acc


def _tc_dense(p, rcp, x, wlT, wrT, b2, relu):
    body = functools.partial(_dense_body, relu)
    return pl.pallas_call(
        body,
        grid=(NBLK,),
        in_specs=[
            pl.BlockSpec((NC, BLK, D), lambda i: (0, i, 0)),
            pl.BlockSpec((BLK, L), lambda i: (i, 0)),
            pl.BlockSpec((BLK, D), lambda i: (i, 0)),
            pl.BlockSpec((D, D), lambda i: (0, 0)),
            pl.BlockSpec((D, D), lambda i: (0, 0)),
            pl.BlockSpec((1, D), lambda i: (0, 0)),
        ],
        out_specs=pl.BlockSpec((BLK, D), lambda i: (i, 0)),
        out_shape=jax.ShapeDtypeStruct((N, D), jnp.float32),
    )(p, rcp, x, wlT, wrT, b2)


def kernel(x, edge_index, Wl0, bl0, Wr0, Wl1, bl1, Wr1, Wl2, bl2, Wr2):
    ei = edge_index.reshape(-1)
    c2 = _sc_count(ei)
    p = _sc_aggregate(x, ei)
    rcp = _tc_rcp(c2)
    h = _tc_dense(p, rcp, x, Wl0, Wr0, bl0.reshape(1, D), True)
    p = _sc_aggregate(h, ei)
    h = _tc_dense(p, rcp, h, Wl1, Wr1, bl1.reshape(1, D), True)
    p = _sc_aggregate(h, ei)
    h = _tc_dense(p, rcp, h, Wl2, Wr2, bl2.reshape(1, D), False)
    return h
